# Initial kernel scaffold; baseline (speedup 1.0000x reference)
import jax
import jax.numpy as jnp
from jax.experimental import pallas as pl


def kernel(x, edge_index, batch, params):
    raise NotImplementedError("write your pallas kernel here")


# speedup vs baseline: 12.7069x; 12.7069x over previous
import functools

import jax
import jax.numpy as jnp
from jax import lax
from jax.experimental import pallas as pl
from jax.experimental.pallas import tpu as pltpu
from jax.experimental.pallas import tpu_sc as plsc

_N = 50000
_E = 800000
_H = 128
_G = 256
_EPS = 1e-5

_NTILE = 16
_NSC = 2
_CHUNK = 128
_NCH = 400
_EP = _NTILE * _NCH * _CHUNK
_HCH = _NCH // 2
_RPT = 3128
_NR = _NTILE * _RPT

_BNC = 3128
_NBC = _NR // _BNC

_BN = 2000
_NB = _N // _BN


@functools.cache
def _mesh():
    return plsc.VectorSubcoreMesh(core_axis_name="c", subcore_axis_name="s",
                                  num_cores=_NSC, num_subcores=_NTILE)


def _edge_pipeline(table, sidx_v, didx_v, buf_v, acc_sh, gsem, ssem, n):
    for p in range(3):
        pltpu.async_copy(table.at[sidx_v.at[p]], buf_v.at[p], gsem)

    def body(j, carry):
        pltpu.make_async_copy(table.at[sidx_v.at[j]], buf_v.at[j % 4], gsem).wait()
        pltpu.async_copy(buf_v.at[j % 4], acc_sh.at[didx_v.at[j]], ssem, add=True)

        @pl.when((j + 3 < n) & (j >= 1))
        def _():
            pltpu.make_async_copy(buf_v.at[j % 4], acc_sh.at[didx_v.at[j]],
                                  ssem).wait()

        @pl.when(j + 3 < n)
        def _():
            pltpu.async_copy(table.at[sidx_v.at[j + 3]], buf_v.at[(j + 3) % 4],
                             gsem)

        return carry

    lax.fori_loop(0, n, body, 0)
    for _ in range(4):
        pltpu.make_async_copy(buf_v.at[0], acc_sh.at[didx_v.at[0]], ssem).wait()


def _agg4_body(table_hbm, src_hbm, dst_hbm, out_hbm, sidx_v, didx_v, buf_v,
               acc_sh, gsem, ssem):
    cid = lax.axis_index("c")
    sid = lax.axis_index("s")
    base = sid * _RPT
    pltpu.sync_copy(src_hbm.at[sid, pl.ds(cid * _HCH, _HCH)], sidx_v)
    pltpu.sync_copy(dst_hbm.at[sid, pl.ds(cid * _HCH, _HCH)], didx_v)

    pltpu.sync_copy(table_hbm.at[pl.ds(base, _RPT)], acc_sh.at[pl.ds(base, _RPT)])
    plsc.subcore_barrier()
    _edge_pipeline(table_hbm, sidx_v, didx_v, buf_v, acc_sh, gsem, ssem, _HCH)
    plsc.subcore_barrier()
    pltpu.sync_copy(acc_sh.at[pl.ds(base, _RPT)],
                    out_hbm.at[cid, pl.ds(base, _RPT), pl.ds(0, 16)])


@functools.cache
def _agg4_call():
    return pl.kernel(
        _agg4_body,
        mesh=_mesh(),
        compiler_params=pltpu.CompilerParams(use_tc_tiling_on_sc=False),
        out_type=jax.ShapeDtypeStruct((_NSC, _NR, 32), jnp.float32),
        scratch_types=[
            pltpu.VMEM((_HCH, _CHUNK), jnp.int32),
            pltpu.VMEM((_HCH, _CHUNK), jnp.int32),
            pltpu.VMEM((4, _CHUNK, 16), jnp.float32),
            pltpu.VMEM_SHARED((_NR, 16), jnp.float32),
            pltpu.SemaphoreType.DMA,
            pltpu.SemaphoreType.DMA,
        ],
    )


def _agg128_body(table_hbm, src_hbm, dst_hbm, out_hbm, sidx_v, didx_v, buf_v,
                 acc_sh, gsem, ssem):
    cid = lax.axis_index("c")
    sid = lax.axis_index("s")
    base = sid * _RPT

    for k in range(4):
        ci = cid + 2 * k
        pltpu.sync_copy(table_hbm.at[ci, pl.ds(base, _RPT)],
                        acc_sh.at[pl.ds(base, _RPT)])
        plsc.subcore_barrier()

        for h in range(2):
            pltpu.sync_copy(src_hbm.at[sid, pl.ds(h * _HCH, _HCH)], sidx_v)
            pltpu.sync_copy(dst_hbm.at[sid, pl.ds(h * _HCH, _HCH)], didx_v)
            _edge_pipeline(table_hbm.at[ci], sidx_v, didx_v, buf_v, acc_sh,
                           gsem, ssem, _HCH)
        plsc.subcore_barrier()
        pltpu.sync_copy(acc_sh.at[pl.ds(base, _RPT)],
                        out_hbm.at[pl.ds(base, _RPT), pl.ds(ci * 16, 16)])
        plsc.subcore_barrier()


@functools.cache
def _agg128_call():
    return pl.kernel(
        _agg128_body,
        mesh=_mesh(),
        compiler_params=pltpu.CompilerParams(use_tc_tiling_on_sc=False),
        out_type=jax.ShapeDtypeStruct((_NR, 128), jnp.float32),
        scratch_types=[
            pltpu.VMEM((_HCH, _CHUNK), jnp.int32),
            pltpu.VMEM((_HCH, _CHUNK), jnp.int32),
            pltpu.VMEM((4, _CHUNK, 16), jnp.float32),
            pltpu.VMEM_SHARED((_NR, 16), jnp.float32),
            pltpu.SemaphoreType.DMA,
            pltpu.SemaphoreType.DMA,
        ],
    )


def _xstats_body(x_ref, o_ref):
    x = x_ref[...]
    s1 = jnp.sum(x, axis=0, keepdims=True)
    s2 = jnp.sum(x * x, axis=0, keepdims=True)
    pad = jnp.zeros((1, 124), jnp.float32)
    o_ref[...] = jnp.concatenate(
        [jnp.concatenate([s1, pad], axis=1),
         jnp.concatenate([s2, pad], axis=1),
         jnp.zeros((6, 128), jnp.float32)], axis=0)


def _gn_from_stats(t, stats, w, b, ms, n):
    mean = stats[0:1, :] / n
    ex2 = stats[1:2, :] / n
    mm = ms * mean
    var = ex2 - 2.0 * mm * mean + mm * mm
    return w * (t - mm) * lax.rsqrt(var + _EPS) + b


def _row_mask(t, j):
    lim = _N - j * _BNC
    msk = lax.broadcasted_iota(jnp.int32, t.shape, 0) < lim
    return jnp.where(msk, t, 0.0)


def _prep_body(x_ref, dg_ref, st_ref, pv_ref, hp_ref, dis_ref):
    w = pv_ref[0:1, :4]
    b = pv_ref[1:2, :4]
    ms = pv_ref[2:3, :4]
    h0 = _gn_from_stats(x_ref[...], st_ref[...][:, :4], w, b, ms, float(_N))
    d = dg_ref[0][:, 0:1] + dg_ref[1][:, 0:1] - 1.0
    dis = lax.rsqrt(d)
    hp = h0 * dis
    hp_ref[...] = jnp.concatenate([hp, jnp.zeros((_BNC, 12), jnp.float32)], axis=1)
    dis_ref[...] = dis


def _conv1_t(agg_ref, hp_ref, dis_ref, w_ref, pv_ref):
    a = agg_ref[0][:, :16] + agg_ref[1][:, :16] - hp_ref[...]
    t = jnp.dot(a * dis_ref[...], w_ref[...],
                preferred_element_type=jnp.float32) + pv_ref[0:1, :]
    return jnp.maximum(t, 0.0)


def _conv1_stats_body(agg_ref, hp_ref, dis_ref, w_ref, pv_ref, st_ref):
    j = pl.program_id(0)

    @pl.when(j == 0)
    def _():
        st_ref[...] = jnp.zeros((8, 128), jnp.float32)

    t = _row_mask(_conv1_t(agg_ref, hp_ref, dis_ref, w_ref, pv_ref), j)
    st_ref[0:1, :] += jnp.sum(t, axis=0, keepdims=True)
    st_ref[1:2, :] += jnp.sum(t * t, axis=0, keepdims=True)


def _conv1_apply_body(agg_ref, hp_ref, dis_ref, w_ref, pv_ref, st_ref, o_ref):
    t = _conv1_t(agg_ref, hp_ref, dis_ref, w_ref, pv_ref)
    g = _gn_from_stats(t, st_ref[...], pv_ref[1:2, :], pv_ref[2:3, :],
                       pv_ref[3:4, :], float(_N))
    o_ref[...] = g * dis_ref[...]


def _conv_t(agg_ref, dis_ref, w_ref, pv_ref):
    t = jnp.dot(agg_ref[...] * dis_ref[...], w_ref[...],
                preferred_element_type=jnp.float32) + pv_ref[0:1, :]
    return jnp.maximum(t, 0.0)


def _conv_stats_body(agg_ref, dis_ref, w_ref, pv_ref, st_ref):
    j = pl.program_id(0)

    @pl.when(j == 0)
    def _():
        st_ref[...] = jnp.zeros((8, 128), jnp.float32)

    t = _row_mask(_conv_t(agg_ref, dis_ref, w_ref, pv_ref), j)
    st_ref[0:1, :] += jnp.sum(t, axis=0, keepdims=True)
    st_ref[1:2, :] += jnp.sum(t * t, axis=0, keepdims=True)


def _conv_apply_body(agg_ref, dis_ref, w_ref, pv_ref, st_ref, o_ref):
    t = _conv_t(agg_ref, dis_ref, w_ref, pv_ref)
    g = _gn_from_stats(t, st_ref[...], pv_ref[1:2, :], pv_ref[2:3, :],
                       pv_ref[3:4, :], float(_N))
    o_ref[...] = g * dis_ref[...]


def _conv3_apply_body(agg_ref, dis_ref, w_ref, pv_ref, st_ref, o_ref):
    t = _conv_t(agg_ref, dis_ref, w_ref, pv_ref)
    o_ref[...] = _gn_from_stats(t, st_ref[...], pv_ref[1:2, :], pv_ref[2:3, :],
                                pv_ref[3:4, :], float(_N))


def _pool_body(h_ref, b_ref, gw1_ref, gw2_ref, gw3_ref, lw1_ref, lw2_ref,
               ow_ref, pv_ref, o_ref, accp, accs):
    j = pl.program_id(0)

    @pl.when(j == 0)
    def _():
        accp[...] = jnp.zeros((_G, 128), jnp.float32)
        accs[...] = jnp.zeros((_G, 1), jnp.float32)

    h = h_ref[...]
    g1 = jnp.maximum(jnp.dot(h, gw1_ref[...],
                             preferred_element_type=jnp.float32) + pv_ref[0:1, :], 0.0)
    g2 = jnp.maximum(jnp.dot(g1, gw2_ref[...],
                             preferred_element_type=jnp.float32) + pv_ref[1:2, :], 0.0)
    gate = jnp.dot(g2, gw3_ref[...],
                   preferred_element_type=jnp.float32) + pv_ref[2:3, 0:1]
    e = jnp.exp(gate)
    gids = lax.broadcasted_iota(jnp.int32, (_BN, _G), 1)
    oh = (gids == b_ref[...]).astype(jnp.float32)
    dn = (((0,), (0,)), ((), ()))
    accp[...] += lax.dot_general(oh, e * h, dn, preferred_element_type=jnp.float32)
    accs[...] += lax.dot_general(oh, e, dn, preferred_element_type=jnp.float32)

    @pl.when(j == _NB - 1)
    def _():
        pooled = accp[...] / (accs[...] + 1e-16)
        z1 = jnp.maximum(jnp.dot(pooled, lw1_ref[...],
                                 preferred_element_type=jnp.float32) + pv_ref[3:4, :], 0.0)
        z2 = jnp.maximum(jnp.dot(z1, lw2_ref[...],
                                 preferred_element_type=jnp.float32) + pv_ref[4:5, :], 0.0)
        o_ref[...] = jnp.dot(z2, ow_ref[...],
                             preferred_element_type=jnp.float32) + pv_ref[5:6, 0:1]


def _pack_rows(rows):
    pv = jnp.zeros((8, 128), jnp.float32)
    for i, r in enumerate(rows):
        r = jnp.asarray(r, jnp.float32).reshape(-1)
        pv = pv.at[i, :r.shape[0]].set(r)
    return pv


def _full(shape):
    return pl.BlockSpec(shape, lambda j: tuple(0 for _ in shape))


def _chunked(h):
    return h.reshape(_NR, 8, 16).transpose(1, 0, 2)


_DSPEC = pl.BlockSpec((_BNC, 1), lambda j: (j, 0))
_NSPEC = pl.BlockSpec((_BNC, 128), lambda j: (j, 0))


def kernel(x, edge_index, batch, params):
    p = params
    pad = _EP - _E
    srcp = jnp.concatenate([edge_index[0], jnp.arange(pad, dtype=jnp.int32)])
    dstp = jnp.concatenate([edge_index[1], jnp.full((pad,), _N, jnp.int32)])
    src3 = srcp.reshape(_NTILE, _NCH, _CHUNK)
    dst3 = dstp.reshape(_NTILE, _NCH, _CHUNK)
    ones = jnp.ones((_NR, 16), jnp.float32)

    degp = _agg4_call()(ones, dst3, dst3)

    xst = pl.pallas_call(
        _xstats_body,
        out_shape=jax.ShapeDtypeStruct((8, 128), jnp.float32),
    )(x)

    hp0, dis = pl.pallas_call(
        _prep_body,
        grid=(_NBC,),
        in_specs=[pl.BlockSpec((_BNC, 4), lambda j: (j, 0)),
                  pl.BlockSpec((_NSC, _BNC, 32), lambda j: (0, j, 0)),
                  _full((8, 128)), _full((8, 128))],
        out_specs=[pl.BlockSpec((_BNC, 16), lambda j: (j, 0)), _DSPEC],
        out_shape=[jax.ShapeDtypeStruct((_NR, 16), jnp.float32),
                   jax.ShapeDtypeStruct((_NR, 1), jnp.float32)],
    )(x, degp, xst, _pack_rows([p["gn0"]["w"], p["gn0"]["b"], p["gn0"]["ms"]]))

    aggp = _agg4_call()(hp0, src3, dst3)

    w1p = jnp.zeros((16, 128), jnp.float32).at[:4].set(p["conv1"]["W"])
    pv1 = _pack_rows([p["conv1"]["b"], p["gn1"]["w"], p["gn1"]["b"], p["gn1"]["ms"]])
    c1_in = [pl.BlockSpec((_NSC, _BNC, 32), lambda j: (0, j, 0)),
             pl.BlockSpec((_BNC, 16), lambda j: (j, 0)), _DSPEC,
             _full((16, 128)), _full((8, 128))]
    st1 = pl.pallas_call(
        _conv1_stats_body,
        grid=(_NBC,),
        in_specs=c1_in,
        out_specs=_full((8, 128)),
        out_shape=jax.ShapeDtypeStruct((8, 128), jnp.float32),
    )(aggp, hp0, dis, w1p, pv1)
    hc = pl.pallas_call(
        _conv1_apply_body,
        grid=(_NBC,),
        in_specs=c1_in + [_full((8, 128))],
        out_specs=_NSPEC,
        out_shape=jax.ShapeDtypeStruct((_NR, 128), jnp.float32),
    )(aggp, hp0, dis, w1p, pv1, st1)

    h3 = None
    for li, (cw, gn, apply_body) in enumerate([
        (p["conv2"], p["gn2"], _conv_apply_body),
        (p["conv3"], p["gn3"], _conv3_apply_body),
    ]):
        agg = _agg128_call()(_chunked(hc), src3, dst3)
        pv = _pack_rows([cw["b"], gn["w"], gn["b"], gn["ms"]])
        cin = [_NSPEC, _DSPEC, _full((128, 128)), _full((8, 128))]
        st = pl.pallas_call(
            _conv_stats_body,
            grid=(_NBC,),
            in_specs=cin,
            out_specs=_full((8, 128)),
            out_shape=jax.ShapeDtypeStruct((8, 128), jnp.float32),
        )(agg, dis, cw["W"], pv)
        if li == 0:
            hc = pl.pallas_call(
                apply_body,
                grid=(_NBC,),
                in_specs=cin + [_full((8, 128))],
                out_specs=_NSPEC,
                out_shape=jax.ShapeDtypeStruct((_NR, 128), jnp.float32),
            )(agg, dis, cw["W"], pv, st)
        else:
            h3 = pl.pallas_call(
                apply_body,
                grid=(_NBC,),
                in_specs=cin + [_full((8, 128))],
                out_specs=_NSPEC,
                out_shape=jax.ShapeDtypeStruct((_N, 128), jnp.float32),
            )(agg, dis, cw["W"], pv, st)

    gp = p["gate"]
    pv4 = _pack_rows([gp["b1"], gp["b2"], gp["b3"], p["lin1"]["b"],
                      p["lin2"]["b"], p["out"]["b"]])
    res = pl.pallas_call(
        _pool_body,
        grid=(_NB,),
        in_specs=[pl.BlockSpec((_BN, 128), lambda j: (j, 0)),
                  pl.BlockSpec((_BN, 1), lambda j: (j, 0)),
                  _full((128, 128)), _full((128, 128)), _full((128, 1)),
                  _full((128, 128)), _full((128, 128)), _full((128, 1)),
                  _full((8, 128))],
        out_specs=_full((_G, 1)),
        out_shape=jax.ShapeDtypeStruct((_G, 1), jnp.float32),
        scratch_shapes=[pltpu.VMEM((_G, 128), jnp.float32),
                        pltpu.VMEM((_G, 1), jnp.float32)],
    )(h3, batch.reshape(_N, 1), gp["W1"], gp["W2"], gp["W3"],
      p["lin1"]["W"], p["lin2"]["W"], p["out"]["W"], pv4)
    return res


# speedup vs baseline: 16.6702x; 1.3119x over previous
import functools

import jax
import jax.numpy as jnp
from jax import lax
from jax.experimental import pallas as pl
from jax.experimental.pallas import tpu as pltpu
from jax.experimental.pallas import tpu_sc as plsc

_N = 50000
_E = 800000
_H = 128
_G = 256
_EPS = 1e-5

_NTILE = 16
_NSC = 2
_CHUNK = 128
_NCH = 400
_EP = _NTILE * _NCH * _CHUNK
_HCH = _NCH // 2
_RPT = 3128
_NR = _NTILE * _RPT

_BNC = 3128
_NBC = _NR // _BNC

_BN = 2000
_NB = _N // _BN


@functools.cache
def _mesh():
    return plsc.VectorSubcoreMesh(core_axis_name="c", subcore_axis_name="s",
                                  num_cores=_NSC, num_subcores=_NTILE)


def _edge_pipeline(table, sidx_v, didx_v, buf_v, acc_sh, gsem, ssem, n):
    for p in range(7):
        pltpu.async_copy(table.at[sidx_v.at[p]], buf_v.at[p], gsem)

    def body(j, carry):
        pltpu.make_async_copy(table.at[sidx_v.at[j]], buf_v.at[j % 8], gsem).wait()
        pltpu.async_copy(buf_v.at[j % 8], acc_sh.at[didx_v.at[j]], ssem, add=True)

        @pl.when((j + 7 < n) & (j >= 1))
        def _():
            pltpu.make_async_copy(buf_v.at[j % 8], acc_sh.at[didx_v.at[j]],
                                  ssem).wait()

        @pl.when(j + 7 < n)
        def _():
            pltpu.async_copy(table.at[sidx_v.at[j + 7]], buf_v.at[(j + 7) % 8],
                             gsem)

        return carry

    lax.fori_loop(0, n, body, 0)
    for _ in range(8):
        pltpu.make_async_copy(buf_v.at[0], acc_sh.at[didx_v.at[0]], ssem).wait()


def _agg4_body(table_hbm, src_hbm, dst_hbm, out_hbm, sidx_v, didx_v, buf_v,
               acc_sh, gsem, ssem):
    cid = lax.axis_index("c")
    sid = lax.axis_index("s")
    base = sid * _RPT
    pltpu.sync_copy(src_hbm.at[sid, pl.ds(cid * _HCH, _HCH)], sidx_v)
    pltpu.sync_copy(dst_hbm.at[sid, pl.ds(cid * _HCH, _HCH)], didx_v)

    pltpu.sync_copy(table_hbm.at[pl.ds(base, _RPT)], acc_sh.at[pl.ds(base, _RPT)])
    plsc.subcore_barrier()
    _edge_pipeline(table_hbm, sidx_v, didx_v, buf_v, acc_sh, gsem, ssem, _HCH)
    plsc.subcore_barrier()
    pltpu.sync_copy(acc_sh.at[pl.ds(base, _RPT)],
                    out_hbm.at[cid, pl.ds(base, _RPT), pl.ds(0, 16)])


@functools.cache
def _agg4_call():
    return pl.kernel(
        _agg4_body,
        mesh=_mesh(),
        compiler_params=pltpu.CompilerParams(use_tc_tiling_on_sc=False),
        out_type=jax.ShapeDtypeStruct((_NSC, _NR, 32), jnp.float32),
        scratch_types=[
            pltpu.VMEM((_HCH, _CHUNK), jnp.int32),
            pltpu.VMEM((_HCH, _CHUNK), jnp.int32),
            pltpu.VMEM((8, _CHUNK, 16), jnp.float32),
            pltpu.VMEM_SHARED((_NR, 16), jnp.float32),
            pltpu.SemaphoreType.DMA,
            pltpu.SemaphoreType.DMA,
        ],
    )


def _agg128_body(table_hbm, src_hbm, dst_hbm, out_hbm, sidx_v, didx_v, buf_v,
                 acc_sh, gsem, ssem):
    cid = lax.axis_index("c")
    sid = lax.axis_index("s")
    base = sid * _RPT

    for k in range(4):
        ci = cid + 2 * k
        pltpu.sync_copy(table_hbm.at[ci, pl.ds(base, _RPT)],
                        acc_sh.at[pl.ds(base, _RPT)])
        plsc.subcore_barrier()

        for h in range(2):
            pltpu.sync_copy(src_hbm.at[sid, pl.ds(h * _HCH, _HCH)], sidx_v)
            pltpu.sync_copy(dst_hbm.at[sid, pl.ds(h * _HCH, _HCH)], didx_v)
            _edge_pipeline(table_hbm.at[ci], sidx_v, didx_v, buf_v, acc_sh,
                           gsem, ssem, _HCH)
        plsc.subcore_barrier()
        pltpu.sync_copy(acc_sh.at[pl.ds(base, _RPT)],
                        out_hbm.at[pl.ds(base, _RPT), pl.ds(ci * 16, 16)])
        plsc.subcore_barrier()


@functools.cache
def _agg128_call():
    return pl.kernel(
        _agg128_body,
        mesh=_mesh(),
        compiler_params=pltpu.CompilerParams(use_tc_tiling_on_sc=False),
        out_type=jax.ShapeDtypeStruct((_NR, 128), jnp.float32),
        scratch_types=[
            pltpu.VMEM((_HCH, _CHUNK), jnp.int32),
            pltpu.VMEM((_HCH, _CHUNK), jnp.int32),
            pltpu.VMEM((8, _CHUNK, 16), jnp.float32),
            pltpu.VMEM_SHARED((_NR, 16), jnp.float32),
            pltpu.SemaphoreType.DMA,
            pltpu.SemaphoreType.DMA,
        ],
    )


def _xstats_body(x_ref, o_ref):
    x = x_ref[...]
    s1 = jnp.sum(x, axis=0, keepdims=True)
    s2 = jnp.sum(x * x, axis=0, keepdims=True)
    pad = jnp.zeros((1, 124), jnp.float32)
    o_ref[...] = jnp.concatenate(
        [jnp.concatenate([s1, pad], axis=1),
         jnp.concatenate([s2, pad], axis=1),
         jnp.zeros((6, 128), jnp.float32)], axis=0)


def _gn_from_stats(t, stats, w, b, ms, n):
    mean = stats[0:1, :] / n
    ex2 = stats[1:2, :] / n
    mm = ms * mean
    var = ex2 - 2.0 * mm * mean + mm * mm
    return w * (t - mm) * lax.rsqrt(var + _EPS) + b


def _row_mask(t, j):
    lim = _N - j * _BNC
    msk = lax.broadcasted_iota(jnp.int32, t.shape, 0) < lim
    return jnp.where(msk, t, 0.0)


def _prep_body(x_ref, dg_ref, st_ref, pv_ref, hp_ref, dis_ref):
    w = pv_ref[0:1, :4]
    b = pv_ref[1:2, :4]
    ms = pv_ref[2:3, :4]
    h0 = _gn_from_stats(x_ref[...], st_ref[...][:, :4], w, b, ms, float(_N))
    d = dg_ref[0][:, 0:1] + dg_ref[1][:, 0:1] - 1.0
    dis = lax.rsqrt(d)
    hp = h0 * dis
    hp_ref[...] = jnp.concatenate([hp, jnp.zeros((_BNC, 12), jnp.float32)], axis=1)
    dis_ref[...] = dis


def _conv1_t(agg_ref, hp_ref, dis_ref, w_ref, pv_ref):
    a = agg_ref[0][:, :16] + agg_ref[1][:, :16] - hp_ref[...]
    t = jnp.dot(a * dis_ref[...], w_ref[...],
                preferred_element_type=jnp.float32) + pv_ref[0:1, :]
    return jnp.maximum(t, 0.0)


def _conv1_stats_body(agg_ref, hp_ref, dis_ref, w_ref, pv_ref, st_ref):
    j = pl.program_id(0)

    @pl.when(j == 0)
    def _():
        st_ref[...] = jnp.zeros((8, 128), jnp.float32)

    t = _row_mask(_conv1_t(agg_ref, hp_ref, dis_ref, w_ref, pv_ref), j)
    st_ref[0:1, :] += jnp.sum(t, axis=0, keepdims=True)
    st_ref[1:2, :] += jnp.sum(t * t, axis=0, keepdims=True)


def _conv1_apply_body(agg_ref, hp_ref, dis_ref, w_ref, pv_ref, st_ref, o_ref):
    t = _conv1_t(agg_ref, hp_ref, dis_ref, w_ref, pv_ref)
    g = _gn_from_stats(t, st_ref[...], pv_ref[1:2, :], pv_ref[2:3, :],
                       pv_ref[3:4, :], float(_N))
    o_ref[...] = g * dis_ref[...]


def _conv_t(agg_ref, dis_ref, w_ref, pv_ref):
    t = jnp.dot(agg_ref[...] * dis_ref[...], w_ref[...],
                preferred_element_type=jnp.float32) + pv_ref[0:1, :]
    return jnp.maximum(t, 0.0)


def _conv_stats_body(agg_ref, dis_ref, w_ref, pv_ref, st_ref):
    j = pl.program_id(0)

    @pl.when(j == 0)
    def _():
        st_ref[...] = jnp.zeros((8, 128), jnp.float32)

    t = _row_mask(_conv_t(agg_ref, dis_ref, w_ref, pv_ref), j)
    st_ref[0:1, :] += jnp.sum(t, axis=0, keepdims=True)
    st_ref[1:2, :] += jnp.sum(t * t, axis=0, keepdims=True)


def _conv_apply_body(agg_ref, dis_ref, w_ref, pv_ref, st_ref, o_ref):
    t = _conv_t(agg_ref, dis_ref, w_ref, pv_ref)
    g = _gn_from_stats(t, st_ref[...], pv_ref[1:2, :], pv_ref[2:3, :],
                       pv_ref[3:4, :], float(_N))
    o_ref[...] = g * dis_ref[...]


def _conv3_apply_body(agg_ref, dis_ref, w_ref, pv_ref, st_ref, o_ref):
    t = _conv_t(agg_ref, dis_ref, w_ref, pv_ref)
    o_ref[...] = _gn_from_stats(t, st_ref[...], pv_ref[1:2, :], pv_ref[2:3, :],
                                pv_ref[3:4, :], float(_N))


def _pool_body(h_ref, b_ref, gw1_ref, gw2_ref, gw3_ref, lw1_ref, lw2_ref,
               ow_ref, pv_ref, o_ref, accp, accs):
    j = pl.program_id(0)

    @pl.when(j == 0)
    def _():
        accp[...] = jnp.zeros((_G, 128), jnp.float32)
        accs[...] = jnp.zeros((_G, 1), jnp.float32)

    h = h_ref[...]
    g1 = jnp.maximum(jnp.dot(h, gw1_ref[...],
                             preferred_element_type=jnp.float32) + pv_ref[0:1, :], 0.0)
    g2 = jnp.maximum(jnp.dot(g1, gw2_ref[...],
                             preferred_element_type=jnp.float32) + pv_ref[1:2, :], 0.0)
    gate = jnp.dot(g2, gw3_ref[...],
                   preferred_element_type=jnp.float32) + pv_ref[2:3, 0:1]
    e = jnp.exp(gate)
    gids = lax.broadcasted_iota(jnp.int32, (_BN, _G), 1)
    oh = (gids == b_ref[...]).astype(jnp.float32)
    dn = (((0,), (0,)), ((), ()))
    accp[...] += lax.dot_general(oh, e * h, dn, preferred_element_type=jnp.float32)
    accs[...] += lax.dot_general(oh, e, dn, preferred_element_type=jnp.float32)

    @pl.when(j == _NB - 1)
    def _():
        pooled = accp[...] / (accs[...] + 1e-16)
        z1 = jnp.maximum(jnp.dot(pooled, lw1_ref[...],
                                 preferred_element_type=jnp.float32) + pv_ref[3:4, :], 0.0)
        z2 = jnp.maximum(jnp.dot(z1, lw2_ref[...],
                                 preferred_element_type=jnp.float32) + pv_ref[4:5, :], 0.0)
        o_ref[...] = jnp.dot(z2, ow_ref[...],
                             preferred_element_type=jnp.float32) + pv_ref[5:6, 0:1]


def _pack_rows(rows):
    pv = jnp.zeros((8, 128), jnp.float32)
    for i, r in enumerate(rows):
        r = jnp.asarray(r, jnp.float32).reshape(-1)
        pv = pv.at[i, :r.shape[0]].set(r)
    return pv


def _full(shape):
    return pl.BlockSpec(shape, lambda j: tuple(0 for _ in shape))


def _chunked(h):
    return h.reshape(_NR, 8, 16).transpose(1, 0, 2)


_DSPEC = pl.BlockSpec((_BNC, 1), lambda j: (j, 0))
_NSPEC = pl.BlockSpec((_BNC, 128), lambda j: (j, 0))


def kernel(x, edge_index, batch, params):
    p = params
    pad = _EP - _E
    srcp = jnp.concatenate([edge_index[0], jnp.arange(pad, dtype=jnp.int32)])
    dstp = jnp.concatenate([edge_index[1], jnp.full((pad,), _N, jnp.int32)])
    src3 = srcp.reshape(_NTILE, _NCH, _CHUNK)
    dst3 = dstp.reshape(_NTILE, _NCH, _CHUNK)
    ones = jnp.ones((_NR, 16), jnp.float32)

    degp = _agg4_call()(ones, dst3, dst3)

    xst = pl.pallas_call(
        _xstats_body,
        out_shape=jax.ShapeDtypeStruct((8, 128), jnp.float32),
    )(x)

    hp0, dis = pl.pallas_call(
        _prep_body,
        grid=(_NBC,),
        in_specs=[pl.BlockSpec((_BNC, 4), lambda j: (j, 0)),
                  pl.BlockSpec((_NSC, _BNC, 32), lambda j: (0, j, 0)),
                  _full((8, 128)), _full((8, 128))],
        out_specs=[pl.BlockSpec((_BNC, 16), lambda j: (j, 0)), _DSPEC],
        out_shape=[jax.ShapeDtypeStruct((_NR, 16), jnp.float32),
                   jax.ShapeDtypeStruct((_NR, 1), jnp.float32)],
    )(x, degp, xst, _pack_rows([p["gn0"]["w"], p["gn0"]["b"], p["gn0"]["ms"]]))

    aggp = _agg4_call()(hp0, src3, dst3)

    w1p = jnp.zeros((16, 128), jnp.float32).at[:4].set(p["conv1"]["W"])
    pv1 = _pack_rows([p["conv1"]["b"], p["gn1"]["w"], p["gn1"]["b"], p["gn1"]["ms"]])
    c1_in = [pl.BlockSpec((_NSC, _BNC, 32), lambda j: (0, j, 0)),
             pl.BlockSpec((_BNC, 16), lambda j: (j, 0)), _DSPEC,
             _full((16, 128)), _full((8, 128))]
    st1 = pl.pallas_call(
        _conv1_stats_body,
        grid=(_NBC,),
        in_specs=c1_in,
        out_specs=_full((8, 128)),
        out_shape=jax.ShapeDtypeStruct((8, 128), jnp.float32),
    )(aggp, hp0, dis, w1p, pv1)
    hc = pl.pallas_call(
        _conv1_apply_body,
        grid=(_NBC,),
        in_specs=c1_in + [_full((8, 128))],
        out_specs=_NSPEC,
        out_shape=jax.ShapeDtypeStruct((_NR, 128), jnp.float32),
    )(aggp, hp0, dis, w1p, pv1, st1)

    h3 = None
    for li, (cw, gn, apply_body) in enumerate([
        (p["conv2"], p["gn2"], _conv_apply_body),
        (p["conv3"], p["gn3"], _conv3_apply_body),
    ]):
        agg = _agg128_call()(_chunked(hc), src3, dst3)
        pv = _pack_rows([cw["b"], gn["w"], gn["b"], gn["ms"]])
        cin = [_NSPEC, _DSPEC, _full((128, 128)), _full((8, 128))]
        st = pl.pallas_call(
            _conv_stats_body,
            grid=(_NBC,),
            in_specs=cin,
            out_specs=_full((8, 128)),
            out_shape=jax.ShapeDtypeStruct((8, 128), jnp.float32),
        )(agg, dis, cw["W"], pv)
        if li == 0:
            hc = pl.pallas_call(
                apply_body,
                grid=(_NBC,),
                in_specs=cin + [_full((8, 128))],
                out_specs=_NSPEC,
                out_shape=jax.ShapeDtypeStruct((_NR, 128), jnp.float32),
            )(agg, dis, cw["W"], pv, st)
        else:
            h3 = pl.pallas_call(
                apply_body,
                grid=(_NBC,),
                in_specs=cin + [_full((8, 128))],
                out_specs=_NSPEC,
                out_shape=jax.ShapeDtypeStruct((_N, 128), jnp.float32),
            )(agg, dis, cw["W"], pv, st)

    gp = p["gate"]
    pv4 = _pack_rows([gp["b1"], gp["b2"], gp["b3"], p["lin1"]["b"],
                      p["lin2"]["b"], p["out"]["b"]])
    res = pl.pallas_call(
        _pool_body,
        grid=(_NB,),
        in_specs=[pl.BlockSpec((_BN, 128), lambda j: (j, 0)),
                  pl.BlockSpec((_BN, 1), lambda j: (j, 0)),
                  _full((128, 128)), _full((128, 128)), _full((128, 1)),
                  _full((128, 128)), _full((128, 128)), _full((128, 1)),
                  _full((8, 128))],
        out_specs=_full((_G, 1)),
        out_shape=jax.ShapeDtypeStruct((_G, 1), jnp.float32),
        scratch_shapes=[pltpu.VMEM((_G, 128), jnp.float32),
                        pltpu.VMEM((_G, 1), jnp.float32)],
    )(h3, batch.reshape(_N, 1), gp["W1"], gp["W2"], gp["W3"],
      p["lin1"]["W"], p["lin2"]["W"], p["out"]["W"], pv4)
    return res
